# Initial kernel scaffold; baseline (speedup 1.0000x reference)
import jax
import jax.numpy as jnp
from jax.experimental import pallas as pl


def kernel(segm_logit, side_logit0, side_logit1, center_point, side_points0, side_points1):
    raise NotImplementedError("write your pallas kernel here")


# speedup vs baseline: 1.0929x; 1.0929x over previous
import jax
import jax.numpy as jnp
from jax.experimental import pallas as pl
from jax.experimental.pallas import tpu as pltpu

_K = 512
_STEPS = 16
_LMAX = 40
_CAP = _STEPS + 2


def _peaks_body(x_ref, score_ref, bcls_ref, argc_ref):
    x = x_ref[0]
    C, H, W = x.shape
    m = x[0]
    for c in range(1, C):
        m = jnp.maximum(m, x[c])
    es = []
    s = None
    for c in range(C):
        e = jnp.exp(x[c] - m)
        es.append(e)
        s = e if s is None else s + e
    inv = 1.0 / s
    ninf = jnp.float32(-jnp.inf)
    best = bcls = abest = agc = None
    for c in range(C):
        p = es[c] * inv
        lf = jnp.concatenate([p[:, 1:], jnp.full((H, 1), ninf, p.dtype)], axis=1)
        rt = jnp.concatenate([jnp.full((H, 1), ninf, p.dtype), p[:, :-1]], axis=1)
        mw = jnp.maximum(p, jnp.maximum(lf, rt))
        up = jnp.concatenate([mw[1:], jnp.full((1, W), ninf, p.dtype)], axis=0)
        dn = jnp.concatenate([jnp.full((1, W), ninf, p.dtype), mw[:-1]], axis=0)
        lm = jnp.maximum(mw, jnp.maximum(up, dn))
        pk = (p == lm) & (p > 0.5)
        msk = jnp.where(pk, p, 0.0)
        xc = x[c]
        if best is None:
            best = msk
            bcls = jnp.zeros(msk.shape, jnp.int32)
            abest = xc
            agc = jnp.zeros(msk.shape, jnp.int32)
        else:
            u = msk > best
            bcls = jnp.where(u, c, bcls)
            best = jnp.where(u, msk, best)
            au = xc > abest
            agc = jnp.where(au, c, agc)
            abest = jnp.where(au, xc, abest)
    score_ref[0] = best
    bcls_ref[0] = bcls
    argc_ref[0] = agc


def _dense(xt):
    B, C, H, W = xt.shape
    return pl.pallas_call(
        _peaks_body,
        grid=(B,),
        in_specs=[pl.BlockSpec((1, C, H, W), lambda b: (b, 0, 0, 0))],
        out_specs=[
            pl.BlockSpec((1, H, W), lambda b: (b, 0, 0)),
            pl.BlockSpec((1, H, W), lambda b: (b, 0, 0)),
            pl.BlockSpec((1, H, W), lambda b: (b, 0, 0)),
        ],
        out_shape=[
            jax.ShapeDtypeStruct((B, H, W), jnp.float32),
            jax.ShapeDtypeStruct((B, H, W), jnp.int32),
            jax.ShapeDtypeStruct((B, H, W), jnp.int32),
        ],
    )(xt)


def _side_probe(gp, cid, argcls_b, sp_map, sl_map, H, W):
    gi = jnp.floor(gp).astype(jnp.int32)
    in_g = (gi[0] >= 0) & (gi[0] < H) & (gi[1] >= 0) & (gi[1] < W)
    g0 = jnp.clip(gi[0], 0, H - 1)
    g1 = jnp.clip(gi[1], 0, W - 1)
    sp = gp + 0.5 + sp_map[g0, g1]
    si = jnp.floor(sp).astype(jnp.int32)
    in_s = (si[0] >= 0) & (si[0] < H) & (si[1] >= 0) & (si[1] < W)
    s0 = jnp.clip(si[0], 0, H - 1)
    s1 = jnp.clip(si[1], 0, W - 1)
    ok = in_g & in_s & (argcls_b[s0, s1] == cid) & (sl_map[s0, s1] <= 0.0)
    return sp, ok


def _grow(pts, n, alive, cid, argcls_b, sp_maps, sl_maps, H, W):
    last = pts[jnp.clip(n - 1, 0, _CAP - 1)]
    prev = pts[jnp.clip(n - 2, 0, _CAP - 1)]
    direction = last - prev
    lp, lok = _side_probe(last, cid, argcls_b, sp_maps[0], sl_maps[0], H, W)
    rp, rok = _side_probe(last, cid, argcls_b, sp_maps[1], sl_maps[1], H, W)
    ld = jnp.where(lok, lp[0] * direction[0] + lp[1] * direction[1], -1.0)
    rd = jnp.where(rok, rp[0] * direction[0] + rp[1] * direction[1], -1.0)
    take_l = lok & (ld > 0) & (ld > rd)
    take_r = rok & (rd > 0) & (rd > ld)
    appended = take_l | take_r
    do_append = alive & appended
    new_pt = jnp.where(take_l, lp, rp)
    slot = jnp.clip(n, 0, _CAP - 1)
    pts = pts.at[slot].set(jnp.where(do_append, new_pt, pts[slot]))
    n = n + jnp.where(do_append, 1, 0).astype(jnp.int32)
    alive = alive & appended
    return pts, n, alive


def _trace(ys, xs, cls, valid, argcls, cp, sp0, sp1, sl0, sl1, H, W):
    def trace_peak(py, px, cid, keep, argcls_b, cp_b, sp0_b, sp1_b, sl0_b, sl1_b):
        gp = jnp.stack([py, px]).astype(jnp.float32)
        start = gp + cp_b[py, px]
        lpt, lok = _side_probe(gp, cid, argcls_b, sp0_b, sl0_b, H, W)
        rpt, rok = _side_probe(gp, cid, argcls_b, sp1_b, sl1_b, H, W)

        def mk(p0, ok0):
            pts = jnp.zeros((_CAP, 2), jnp.float32)
            pts = pts.at[0].set(start)
            pts = pts.at[1].set(jnp.where(ok0, p0, 0.0))
            n = jnp.where(ok0, 2, 1).astype(jnp.int32)
            return pts, n, ok0

        def body(_, carry):
            (a_pts, a_n, a_al), (b_pts, b_n, b_al) = carry
            left = _grow(a_pts, a_n, a_al, cid, argcls_b, (sp0_b, sp1_b), (sl0_b, sl1_b), H, W)
            right = _grow(b_pts, b_n, b_al, cid, argcls_b, (sp0_b, sp1_b), (sl0_b, sl1_b), H, W)
            return left, right

        (pl_, nl, _), (pr_, nr, _) = jax.lax.fori_loop(
            0, _STEPS, body, (mk(lpt, lok), mk(rpt, rok))
        )
        total = nl + nr - 1
        j = jnp.arange(_LMAX)
        from_l = j < nl
        pt = jnp.where(
            from_l[:, None],
            pl_[jnp.clip(nl - 1 - j, 0, _CAP - 1)],
            pr_[jnp.clip(j - nl + 1, 0, _CAP - 1)],
        )
        line = jnp.where((j < total)[:, None] & keep, pt, 0.0)
        length = jnp.where(keep, total, 0).astype(jnp.int32)
        return line, length

    per_peak = jax.vmap(trace_peak, in_axes=(0, 0, 0, 0, None, None, None, None, None, None))
    per_batch = jax.vmap(per_peak)
    return per_batch(ys, xs, cls, valid, argcls, cp, sp0, sp1, sl0, sl1)


def kernel(segm_logit, side_logit0, side_logit1, center_point, side_points0, side_points1):
    B, H, W, C = segm_logit.shape
    xt = jnp.transpose(segm_logit, (0, 3, 1, 2))
    best_score, best_class, argcls = _dense(xt)

    top_s, top_i = jax.lax.top_k(best_score.reshape(B, H * W), _K)
    ys = top_i // W
    xs = top_i % W
    cls = jnp.take_along_axis(best_class.reshape(B, H * W), top_i, axis=1)
    valid = top_s > 0.0

    sl0 = side_logit0[..., 0]
    sl1 = side_logit1[..., 0]
    lines, lens = _trace(
        ys, xs, cls, valid, argcls, center_point, side_points0, side_points1, sl0, sl1, H, W
    )
    points = jnp.stack([ys, xs], axis=-1).astype(jnp.int32)
    return (points, cls.astype(jnp.int32), top_s, lines, lens)
